# Initial kernel scaffold; baseline (speedup 1.0000x reference)
import jax
import jax.numpy as jnp
from jax.experimental import pallas as pl


def kernel(input_ids, position_ids, pretrain_embed, subword_table, m1_table, m2_table):
    raise NotImplementedError("write your pallas kernel here")


# speedup vs baseline: 2.9697x; 2.9697x over previous
import functools

import jax
import jax.numpy as jnp
from jax import lax
from jax.experimental import pallas as pl
from jax.experimental.pallas import tpu as pltpu
from jax.experimental.pallas import tpu_sc as plsc

HID = 64
M1, M2 = 47, 11
LMAX = 50
NC, NS, LANES = 2, 16, 16
NW = NC * NS
CHUNK = 128


def _embed(ids, pos, subword_table, m1_table, m2_table):
    n = ids.shape[0]
    n_per_w = n // NW
    n_chunks = n_per_w // CHUNK
    mesh = plsc.VectorSubcoreMesh(
        core_axis_name="c", subcore_axis_name="s", num_cores=NC, num_subcores=NS
    )

    @functools.partial(
        pl.kernel,
        out_type=jax.ShapeDtypeStruct((n, HID), jnp.float32),
        mesh=mesh,
        compiler_params=pltpu.CompilerParams(use_tc_tiling_on_sc=False),
        scratch_types=[
            pltpu.VMEM((CHUNK,), jnp.int32),
            pltpu.VMEM((CHUNK,), jnp.int32),
            pltpu.VMEM((CHUNK, HID), jnp.float32),
            pltpu.VMEM((M1, HID), jnp.float32),
            pltpu.VMEM((M2, HID), jnp.float32),
            pltpu.VMEM((LMAX, HID), jnp.float32),
            pltpu.VMEM_SHARED((LMAX, HID), jnp.float32),
            pltpu.SemaphoreType.DMA,
        ],
    )
    def run(ids_hbm, pos_hbm, sub_hbm, m1_hbm, m2_hbm, out_hbm,
            idx_v, pidx_v, rows_v, m1_v, m2_v, fused_v, fused_sh, sem):
        cid = lax.axis_index("c")
        sid = lax.axis_index("s")

        @pl.when(sid == 0)
        def _build():
            pltpu.sync_copy(m1_hbm, m1_v)
            pltpu.sync_copy(m2_hbm, m2_v)
            for p in range(LMAX):
                for j in range(HID // LANES):
                    sl = pl.ds(j * LANES, LANES)
                    fused_v[p, sl] = m1_v[p % M1, sl] + m2_v[p % M2, sl]
            pltpu.sync_copy(fused_v, fused_sh)

        plsc.subcore_barrier()

        wid = sid * NC + cid
        base = wid * n_per_w

        def chunk_body(i, carry):
            off = base + i * CHUNK
            pltpu.sync_copy(ids_hbm.at[pl.ds(off, CHUNK)], idx_v)
            pltpu.sync_copy(pos_hbm.at[pl.ds(off, CHUNK)], pidx_v)
            pltpu.sync_copy(fused_sh.at[pidx_v], rows_v)
            pltpu.async_copy(sub_hbm.at[idx_v], rows_v, sem, add=True).wait()
            pltpu.sync_copy(rows_v, out_hbm.at[pl.ds(off, CHUNK)])
            return carry

        lax.fori_loop(0, n_chunks, chunk_body, 0)

    return run(ids, pos, subword_table, m1_table, m2_table)


def kernel(input_ids, position_ids, pretrain_embed, subword_table, m1_table, m2_table):
    b, l = input_ids.shape
    n = b * l
    ids = input_ids.reshape(n).astype(jnp.int32)
    pos = position_ids.reshape(n).astype(jnp.int32)
    out = _embed(ids, pos, subword_table, m1_table, m2_table)
    return out.reshape(b, l, HID)


# speedup vs baseline: 3.3415x; 1.1252x over previous
import functools

import jax
import jax.numpy as jnp
from jax import lax
from jax.experimental import pallas as pl
from jax.experimental.pallas import tpu as pltpu
from jax.experimental.pallas import tpu_sc as plsc

HID = 64
M1, M2 = 47, 11
LMAX = 50
NC, NS, LANES = 2, 16, 16
NW = NC * NS
CHUNK = 128
NBUF = 5


def _embed(ids, pos, subword_table, m1_table, m2_table):
    n = ids.size
    n_per_w = n // NW
    n_chunks = n_per_w // CHUNK
    n_rounds = n_chunks // NBUF
    mesh = plsc.VectorSubcoreMesh(
        core_axis_name="c", subcore_axis_name="s", num_cores=NC, num_subcores=NS
    )

    @functools.partial(
        pl.kernel,
        out_type=jax.ShapeDtypeStruct((n, HID), jnp.float32),
        mesh=mesh,
        compiler_params=pltpu.CompilerParams(use_tc_tiling_on_sc=False),
        scratch_types=[
            pltpu.VMEM((n_chunks, CHUNK), jnp.int32),
            pltpu.VMEM((n_chunks, CHUNK), jnp.int32),
            pltpu.VMEM((NBUF, CHUNK, HID), jnp.float32),
            pltpu.VMEM((M1, HID), jnp.float32),
            pltpu.VMEM((M2, HID), jnp.float32),
            pltpu.VMEM((LMAX, HID), jnp.float32),
            pltpu.VMEM_SHARED((LMAX, HID), jnp.float32),
            pltpu.SemaphoreType.DMA,
        ] + [pltpu.SemaphoreType.DMA] * (3 * NBUF),
    )
    def run(ids_hbm, pos_hbm, sub_hbm, m1_hbm, m2_hbm, out_hbm,
            ids_v, pos_v, rows, m1_v, m2_v, fused_v, fused_sh,
            sem_i, *sems):
        sem_p = sems[0:NBUF]
        sem_g = sems[NBUF:2 * NBUF]
        sem_o = sems[2 * NBUF:3 * NBUF]
        cid = lax.axis_index("c")
        sid = lax.axis_index("s")
        wid = sid * NC + cid
        base = wid * n_per_w

        cp_ids = pltpu.async_copy(ids_hbm.at[wid], ids_v, sem_i)
        cp_pos = pltpu.async_copy(pos_hbm.at[wid], pos_v, sem_i)

        @pl.when(sid == 0)
        def _build():
            pltpu.sync_copy(m1_hbm, m1_v)
            pltpu.sync_copy(m2_hbm, m2_v)
            for p in range(LMAX):
                for j in range(HID // LANES):
                    sl = pl.ds(j * LANES, LANES)
                    fused_v[p, sl] = m1_v[p % M1, sl] + m2_v[p % M2, sl]
            pltpu.sync_copy(fused_v, fused_sh)

        plsc.subcore_barrier()
        cp_ids.wait()
        cp_pos.wait()

        def start_p(i, b):
            return pltpu.async_copy(fused_sh.at[pos_v.at[i]], rows.at[b],
                                    sem_p[b])

        def wait_p(i, b):
            pltpu.make_async_copy(fused_sh.at[pos_v.at[i]], rows.at[b],
                                  sem_p[b]).wait()

        def start_g(i, b):
            return pltpu.async_copy(sub_hbm.at[ids_v.at[i]], rows.at[b],
                                    sem_g[b], add=True)

        def start_o(i, b):
            return pltpu.async_copy(rows.at[b],
                                    out_hbm.at[pl.ds(base + i * CHUNK, CHUNK)],
                                    sem_o[b])

        def do_round(i0, prime_next):
            gd = []
            for b in range(NBUF):
                wait_p(i0 + b, b)
                gd.append(start_g(i0 + b, b))
            od = []
            for b in range(NBUF):
                gd[b].wait()
                od.append(start_o(i0 + b, b))
            for b in range(NBUF):
                od[b].wait()
                if prime_next:
                    start_p(i0 + NBUF + b, b)

        for b in range(NBUF):
            start_p(b, b)

        def round_body(r, carry):
            do_round(r * NBUF, prime_next=True)
            return carry

        lax.fori_loop(0, n_rounds - 1, round_body, 0)
        do_round((n_rounds - 1) * NBUF, prime_next=False)

    return run(ids, pos, subword_table, m1_table, m2_table)


def kernel(input_ids, position_ids, pretrain_embed, subword_table, m1_table, m2_table):
    b, l = input_ids.shape
    n = b * l
    n_per_w = n // NW
    n_chunks = n_per_w // CHUNK
    ids = input_ids.reshape(NW, n_chunks, CHUNK).astype(jnp.int32)
    pos = position_ids.reshape(NW, n_chunks, CHUNK).astype(jnp.int32)
    out = _embed(ids, pos, subword_table, m1_table, m2_table)
    return out.reshape(b, l, HID)
